# baseline (speedup 1.0000x reference)
import functools

import jax
import jax.numpy as jnp
from jax import lax
from jax.experimental import pallas as pl
from jax.experimental.pallas import tpu as pltpu
from jax.experimental.pallas import tpu_sc as plsc

N = 10000
E = 320000
D = 128

NC = 2
NS = 16
NW = NC * NS
EPW = E // NW
K = 20
NCHUNK = EPW // K
QC = 20
NP = 10240
RPT = NP // NS
CW = 16


def _agg_body(with_counts, *refs):
    if with_counts:
        (y_hbm, src_hbm, dst_hbm, out_hbm, cnt_hbm,
         sidx, didx, rows, acc, g0, g1, g2, g3, g4, g5, g6, g7,
         isem, ssa, ssb, csa, csb, ones, czbuf, cacc) = refs
    else:
        (y_hbm, src_hbm, dst_hbm, out_hbm,
         sidx, didx, rows, acc, g0, g1, g2, g3, g4, g5, g6, g7,
         isem, ssa, ssb) = refs
    gsems = (g0, g1, g2, g3, g4, g5, g6, g7)

    core = lax.axis_index("c")
    sub = lax.axis_index("s")
    wid = core * NS + sub

    pltpu.async_copy(src_hbm.at[wid, pl.ds(0, QC)], sidx.at[0], isem)
    pltpu.async_copy(dst_hbm.at[wid, pl.ds(0, QC)], didx.at[0], isem)

    zero16 = jnp.zeros((16,), jnp.float32)

    for b in range(2):
        def zrow(i, c, _b=b):
            for j in range(D // 16):
                rows[_b, i, pl.ds(j * 16, 16)] = zero16
            return c
        lax.fori_loop(0, K, zrow, 0)

    r0 = sub * RPT
    for t in range(RPT // K):
        @pl.when(r0 + t * K < N)
        def _(_t=t):
            pltpu.sync_copy(rows.at[_t % 2], acc.at[pl.ds(r0 + _t * K, K)])

    if with_counts:
        one16 = jnp.ones((16,), jnp.float32)

        def crow(i, c):
            czbuf[i, :] = zero16
            return c
        lax.fori_loop(0, K, crow, 0)
        for t in range(RPT // K):
            @pl.when(r0 + t * K < N)
            def _(_t=t):
                pltpu.sync_copy(czbuf, cacc.at[pl.ds(r0 + _t * K, K)])

        def orow(i, c):
            ones[i, :] = one16
            return c
        lax.fori_loop(0, K, orow, 0)

    pltpu.make_async_copy(src_hbm.at[wid, pl.ds(0, QC)], sidx.at[0],
                          isem).wait()
    pltpu.make_async_copy(dst_hbm.at[wid, pl.ds(0, QC)], didx.at[0],
                          isem).wait()
    for p in range(8):
        pltpu.async_copy(y_hbm.at[sidx.at[0, p]], rows.at[p], gsems[p])

    plsc.subcore_barrier()

    NB = 10
    AHEAD = 8

    def half(c, gsem, ssem, csem):
        q = lax.div(c, QC)
        r = lax.rem(c, QC)
        qp = lax.rem(q, 2)
        qn = lax.rem(q + 1, 2)
        b0 = lax.rem(c, NB)
        b4 = lax.rem(c + AHEAD, NB)

        pltpu.make_async_copy(y_hbm.at[sidx.at[qp, r]], rows.at[b0],
                              gsem).wait()

        @pl.when(c >= 2)
        def _():
            pltpu.make_async_copy(y_hbm.at[pl.ds(0, K)], rows.at[b0],
                                  ssem).wait()
            if with_counts:
                pltpu.make_async_copy(cnt_hbm.at[0, pl.ds(0, K)], ones,
                                      csem).wait()

        @pl.when((r == 1) & ((q + 1) * QC < NCHUNK))
        def _():
            pltpu.async_copy(src_hbm.at[wid, pl.ds((q + 1) * QC, QC)],
                             sidx.at[qn], isem)
            pltpu.async_copy(dst_hbm.at[wid, pl.ds((q + 1) * QC, QC)],
                             didx.at[qn], isem)

        @pl.when((r == QC - AHEAD) & (c + AHEAD < NCHUNK))
        def _():
            pltpu.make_async_copy(src_hbm.at[wid, pl.ds(0, QC)],
                                  sidx.at[qn], isem).wait()
            pltpu.make_async_copy(dst_hbm.at[wid, pl.ds(0, QC)],
                                  didx.at[qn], isem).wait()

        @pl.when(c + AHEAD < NCHUNK)
        def _():
            q2 = lax.div(c + AHEAD, QC)
            r2 = lax.rem(c + AHEAD, QC)
            qp2 = lax.rem(q2, 2)
            pltpu.async_copy(y_hbm.at[sidx.at[qp2, r2]], rows.at[b4], gsem)

        pltpu.async_copy(rows.at[b0], acc.at[didx.at[qp, r]], ssem, add=True)
        if with_counts:
            pltpu.async_copy(ones, cacc.at[didx.at[qp, r]], csem, add=True)

    _csa = csa if with_counts else None
    _csb = csb if with_counts else None

    def oct(i, carry):
        for p in range(8):
            half(8 * i + p, gsems[p], ssa if p % 2 == 0 else ssb,
                 _csa if p % 2 == 0 else _csb)
        return carry
    lax.fori_loop(0, NCHUNK // 8, oct, 0)
    for p in range(NCHUNK % 8):
        c_tail = (NCHUNK // 8) * 8 + p
        half(c_tail, gsems[c_tail % 8], ssa if p % 2 == 0 else ssb,
             _csa if p % 2 == 0 else _csb)

    for ss, cs in ((ssa, _csa), (ssb, _csb)):
        pltpu.make_async_copy(y_hbm.at[pl.ds(0, K)], rows.at[0], ss).wait()
        if with_counts:
            pltpu.make_async_copy(cnt_hbm.at[0, pl.ds(0, K)], ones, cs).wait()

    plsc.subcore_barrier()

    LAST = N - (NS - 1) * RPT

    @pl.when(sub < NS - 1)
    def _():
        pltpu.sync_copy(acc.at[pl.ds(r0, RPT)],
                        out_hbm.at[core, pl.ds(r0, RPT)])
        if with_counts:
            pltpu.sync_copy(cacc.at[pl.ds(r0, RPT)],
                            cnt_hbm.at[core, pl.ds(r0, RPT)])

    @pl.when(sub == NS - 1)
    def _():
        pltpu.sync_copy(acc.at[pl.ds((NS - 1) * RPT, LAST)],
                        out_hbm.at[core, pl.ds((NS - 1) * RPT, LAST)])
        if with_counts:
            pltpu.sync_copy(cacc.at[pl.ds((NS - 1) * RPT, LAST)],
                            cnt_hbm.at[core, pl.ds((NS - 1) * RPT, LAST)])


def _make_agg(with_counts):
    mesh = plsc.VectorSubcoreMesh(core_axis_name="c", subcore_axis_name="s")
    out_type = [jax.ShapeDtypeStruct((NC, NP, D), jnp.float32)]
    scratch = [
        pltpu.VMEM((2, QC, K), jnp.int32),
        pltpu.VMEM((2, QC, K), jnp.int32),
        pltpu.VMEM((10, K, D), jnp.float32),
        pltpu.VMEM_SHARED((NP, D), jnp.float32),
    ] + [pltpu.SemaphoreType.DMA] * 8 + [
        pltpu.SemaphoreType.DMA,
        pltpu.SemaphoreType.DMA,
        pltpu.SemaphoreType.DMA,
    ]
    if with_counts:
        out_type.append(jax.ShapeDtypeStruct((NC, NP, CW), jnp.float32))
        scratch += [
            pltpu.SemaphoreType.DMA,
            pltpu.SemaphoreType.DMA,
            pltpu.VMEM((K, CW), jnp.float32),
            pltpu.VMEM((K, CW), jnp.float32),
            pltpu.VMEM_SHARED((NP, CW), jnp.float32),
        ]
    return pl.kernel(
        functools.partial(_agg_body, with_counts),
        out_type=out_type,
        mesh=mesh,
        scratch_types=scratch,
        compiler_params=pltpu.CompilerParams(use_tc_tiling_on_sc=False),
    )


_agg_with_counts = _make_agg(True)
_agg_no_counts = _make_agg(False)


_RB = 1000
_NG = N // _RB

_full_w = pl.BlockSpec((D, D), lambda i: (0, 0))
_full_b = pl.BlockSpec((1, D), lambda i: (0, 0))
_row_blk = pl.BlockSpec((_RB, D), lambda i: (i, 0))
_agg_blk = pl.BlockSpec((NC, _RB, D), lambda i: (0, i, 0))
_cnt_blk = pl.BlockSpec((NC, _RB, CW), lambda i: (0, i, 0))


def _pre_body(x_ref, wl_ref, wr_ref, b_ref, y_ref, s_ref):
    xb = x_ref[...]
    y_ref[...] = jnp.dot(xb, wl_ref[...], preferred_element_type=jnp.float32)
    s_ref[...] = (jnp.dot(xb, wr_ref[...], preferred_element_type=jnp.float32)
                  + b_ref[...])


def _pre(x, wl, wr, b):
    return pl.pallas_call(
        _pre_body,
        grid=(_NG,),
        in_specs=[_row_blk, _full_w, _full_w, _full_b],
        out_specs=[_row_blk, _row_blk],
        out_shape=[jax.ShapeDtypeStruct((N, D), jnp.float32),
                   jax.ShapeDtypeStruct((N, D), jnp.float32)],
    )(x, wl, wr, b)


def _mid_body(agg_ref, cnt_ref, s_ref, wl_ref, wr_ref, b_ref, y_ref, s2_ref):
    a = agg_ref[0] + agg_ref[1]
    cn = cnt_ref[0, :, 0:1] + cnt_ref[1, :, 0:1]
    rinv = 1.0 / jnp.maximum(cn, 1.0)
    z = jnp.maximum(a * rinv + s_ref[...], 0.0)
    y_ref[...] = jnp.dot(z, wl_ref[...], preferred_element_type=jnp.float32)
    s2_ref[...] = (jnp.dot(z, wr_ref[...], preferred_element_type=jnp.float32)
                   + b_ref[...])


def _mid(agg, cnt, s1, wl, wr, b):
    return pl.pallas_call(
        _mid_body,
        grid=(_NG,),
        in_specs=[_agg_blk, _cnt_blk, _row_blk, _full_w, _full_w, _full_b],
        out_specs=[_row_blk, _row_blk],
        out_shape=[jax.ShapeDtypeStruct((N, D), jnp.float32),
                   jax.ShapeDtypeStruct((N, D), jnp.float32)],
    )(agg, cnt, s1, wl, wr, b)


def _fin_body(agg_ref, cnt_ref, s_ref, o_ref):
    a = agg_ref[0] + agg_ref[1]
    cn = cnt_ref[0, :, 0:1] + cnt_ref[1, :, 0:1]
    rinv = 1.0 / jnp.maximum(cn, 1.0)
    o_ref[...] = a * rinv + s_ref[...]


def _fin(agg, cnt, s2):
    return pl.pallas_call(
        _fin_body,
        grid=(_NG,),
        in_specs=[_agg_blk, _cnt_blk, _row_blk],
        out_specs=_row_blk,
        out_shape=jax.ShapeDtypeStruct((N, D), jnp.float32),
    )(agg, cnt, s2)


@jax.jit
def kernel(x, edge_index, W_l1, b_l1, W_r1, W_l2, b_l2, W_r2):
    src = edge_index[0].reshape(NW, NCHUNK, K)
    dst = edge_index[1].reshape(NW, NCHUNK, K)
    y1, s1 = _pre(x, W_l1, W_r1, b_l1.reshape(1, D))
    agg1, cnt = _agg_with_counts(y1, src, dst)
    y2, s2 = _mid(agg1, cnt, s1, W_l2, W_r2, b_l2.reshape(1, D))
    (agg2,) = _agg_no_counts(y2, src, dst)
    return _fin(agg2, cnt, s2)


# speedup vs baseline: 1.1834x; 1.1834x over previous
import functools

import jax
import jax.numpy as jnp
from jax import lax
from jax.experimental import pallas as pl
from jax.experimental.pallas import tpu as pltpu
from jax.experimental.pallas import tpu_sc as plsc

N = 10000
E = 320000
D = 128

NC = 2
NS = 16
NW = NC * NS
EPW = E // NW
K = 40
NCHUNK = EPW // K
QC = 25
NP = 10240
RPT = NP // NS
CW = 16


def _agg_body(with_counts, ahead, nb, *refs):
    if with_counts:
        (y_hbm, src_hbm, dst_hbm, out_hbm, cnt_hbm,
         sidx, didx, rows, acc) = refs[:9]
        gsems = refs[9:9 + ahead]
        (isem, ssa, ssb, csa, csb, ones, czbuf, cacc) = refs[9 + ahead:]
    else:
        (y_hbm, src_hbm, dst_hbm, out_hbm,
         sidx, didx, rows, acc) = refs[:8]
        gsems = refs[8:8 + ahead]
        (isem, ssa, ssb) = refs[8 + ahead:]

    core = lax.axis_index("c")
    sub = lax.axis_index("s")
    wid = core * NS + sub

    pltpu.async_copy(src_hbm.at[wid, pl.ds(0, QC)], sidx.at[0], isem)
    pltpu.async_copy(dst_hbm.at[wid, pl.ds(0, QC)], didx.at[0], isem)

    zero16 = jnp.zeros((16,), jnp.float32)

    for b in range(2):
        def zrow(i, c, _b=b):
            for j in range(D // 16):
                rows[_b, i, pl.ds(j * 16, 16)] = zero16
            return c
        lax.fori_loop(0, K, zrow, 0)

    r0 = sub * RPT
    for t in range(RPT // K):
        @pl.when(r0 + t * K < N)
        def _(_t=t):
            pltpu.sync_copy(rows.at[_t % 2], acc.at[pl.ds(r0 + _t * K, K)])

    if with_counts:
        one16 = jnp.ones((16,), jnp.float32)

        def crow(i, c):
            czbuf[i, :] = zero16
            return c
        lax.fori_loop(0, K, crow, 0)
        for t in range(RPT // K):
            @pl.when(r0 + t * K < N)
            def _(_t=t):
                pltpu.sync_copy(czbuf, cacc.at[pl.ds(r0 + _t * K, K)])

        def orow(i, c):
            ones[i, :] = one16
            return c
        lax.fori_loop(0, K, orow, 0)

    pltpu.make_async_copy(src_hbm.at[wid, pl.ds(0, QC)], sidx.at[0],
                          isem).wait()
    pltpu.make_async_copy(dst_hbm.at[wid, pl.ds(0, QC)], didx.at[0],
                          isem).wait()
    for p in range(ahead):
        pltpu.async_copy(y_hbm.at[sidx.at[0, p]], rows.at[p], gsems[p])

    plsc.subcore_barrier()

    NB = nb
    AHEAD = ahead

    def half(c, gsem, ssem, csem):
        q = lax.div(c, QC)
        r = lax.rem(c, QC)
        qp = lax.rem(q, 2)
        qn = lax.rem(q + 1, 2)
        b0 = lax.rem(c, NB)
        b4 = lax.rem(c + AHEAD, NB)

        pltpu.make_async_copy(y_hbm.at[sidx.at[qp, r]], rows.at[b0],
                              gsem).wait()

        @pl.when(c >= 2)
        def _():
            pltpu.make_async_copy(y_hbm.at[pl.ds(0, K)], rows.at[b0],
                                  ssem).wait()
            if with_counts:
                pltpu.make_async_copy(cnt_hbm.at[0, pl.ds(0, K)], ones,
                                      csem).wait()

        @pl.when((r == 1) & ((q + 1) * QC < NCHUNK))
        def _():
            pltpu.async_copy(src_hbm.at[wid, pl.ds((q + 1) * QC, QC)],
                             sidx.at[qn], isem)
            pltpu.async_copy(dst_hbm.at[wid, pl.ds((q + 1) * QC, QC)],
                             didx.at[qn], isem)

        @pl.when((r == QC - AHEAD) & (c + AHEAD < NCHUNK))
        def _():
            pltpu.make_async_copy(src_hbm.at[wid, pl.ds(0, QC)],
                                  sidx.at[qn], isem).wait()
            pltpu.make_async_copy(dst_hbm.at[wid, pl.ds(0, QC)],
                                  didx.at[qn], isem).wait()

        @pl.when(c + AHEAD < NCHUNK)
        def _():
            q2 = lax.div(c + AHEAD, QC)
            r2 = lax.rem(c + AHEAD, QC)
            qp2 = lax.rem(q2, 2)
            pltpu.async_copy(y_hbm.at[sidx.at[qp2, r2]], rows.at[b4], gsem)

        pltpu.async_copy(rows.at[b0], acc.at[didx.at[qp, r]], ssem, add=True)
        if with_counts:
            pltpu.async_copy(ones, cacc.at[didx.at[qp, r]], csem, add=True)

    _csa = csa if with_counts else None
    _csb = csb if with_counts else None

    def group(i, carry):
        for p in range(AHEAD):
            half(AHEAD * i + p, gsems[p], ssa if p % 2 == 0 else ssb,
                 _csa if p % 2 == 0 else _csb)
        return carry
    lax.fori_loop(0, NCHUNK // AHEAD, group, 0)
    for p in range(NCHUNK % AHEAD):
        c_tail = (NCHUNK // AHEAD) * AHEAD + p
        half(c_tail, gsems[c_tail % AHEAD], ssa if p % 2 == 0 else ssb,
             _csa if p % 2 == 0 else _csb)

    for ss, cs in ((ssa, _csa), (ssb, _csb)):
        pltpu.make_async_copy(y_hbm.at[pl.ds(0, K)], rows.at[0], ss).wait()
        if with_counts:
            pltpu.make_async_copy(cnt_hbm.at[0, pl.ds(0, K)], ones, cs).wait()

    plsc.subcore_barrier()

    LAST = N - (NS - 1) * RPT

    @pl.when(sub < NS - 1)
    def _():
        pltpu.sync_copy(acc.at[pl.ds(r0, RPT)],
                        out_hbm.at[core, pl.ds(r0, RPT)])
        if with_counts:
            pltpu.sync_copy(cacc.at[pl.ds(r0, RPT)],
                            cnt_hbm.at[core, pl.ds(r0, RPT)])

    @pl.when(sub == NS - 1)
    def _():
        pltpu.sync_copy(acc.at[pl.ds((NS - 1) * RPT, LAST)],
                        out_hbm.at[core, pl.ds((NS - 1) * RPT, LAST)])
        if with_counts:
            pltpu.sync_copy(cacc.at[pl.ds((NS - 1) * RPT, LAST)],
                            cnt_hbm.at[core, pl.ds((NS - 1) * RPT, LAST)])


def _make_agg(with_counts, ahead, nb):
    mesh = plsc.VectorSubcoreMesh(core_axis_name="c", subcore_axis_name="s")
    out_type = [jax.ShapeDtypeStruct((NC, NP, D), jnp.float32)]
    scratch = [
        pltpu.VMEM((2, QC, K), jnp.int32),
        pltpu.VMEM((2, QC, K), jnp.int32),
        pltpu.VMEM((nb, K, D), jnp.float32),
        pltpu.VMEM_SHARED((NP, D), jnp.float32),
    ] + [pltpu.SemaphoreType.DMA] * ahead + [
        pltpu.SemaphoreType.DMA,
        pltpu.SemaphoreType.DMA,
        pltpu.SemaphoreType.DMA,
    ]
    if with_counts:
        out_type.append(jax.ShapeDtypeStruct((NC, NP, CW), jnp.float32))
        scratch += [
            pltpu.SemaphoreType.DMA,
            pltpu.SemaphoreType.DMA,
            pltpu.VMEM((K, CW), jnp.float32),
            pltpu.VMEM((K, CW), jnp.float32),
            pltpu.VMEM_SHARED((NP, CW), jnp.float32),
        ]
    return pl.kernel(
        functools.partial(_agg_body, with_counts, ahead, nb),
        out_type=out_type,
        mesh=mesh,
        scratch_types=scratch,
        compiler_params=pltpu.CompilerParams(use_tc_tiling_on_sc=False),
    )


_agg_with_counts = _make_agg(True, 4, 6)
_agg_no_counts = _make_agg(False, 6, 8)


_RB = 1000
_NG = N // _RB

_full_w = pl.BlockSpec((D, D), lambda i: (0, 0))
_full_b = pl.BlockSpec((1, D), lambda i: (0, 0))
_row_blk = pl.BlockSpec((_RB, D), lambda i: (i, 0))
_agg_blk = pl.BlockSpec((NC, _RB, D), lambda i: (0, i, 0))
_cnt_blk = pl.BlockSpec((NC, _RB, CW), lambda i: (0, i, 0))


def _pre_body(x_ref, wl_ref, wr_ref, b_ref, y_ref, s_ref):
    xb = x_ref[...]
    y_ref[...] = jnp.dot(xb, wl_ref[...], preferred_element_type=jnp.float32)
    s_ref[...] = (jnp.dot(xb, wr_ref[...], preferred_element_type=jnp.float32)
                  + b_ref[...])


def _pre(x, wl, wr, b):
    return pl.pallas_call(
        _pre_body,
        grid=(_NG,),
        in_specs=[_row_blk, _full_w, _full_w, _full_b],
        out_specs=[_row_blk, _row_blk],
        out_shape=[jax.ShapeDtypeStruct((N, D), jnp.float32),
                   jax.ShapeDtypeStruct((N, D), jnp.float32)],
    )(x, wl, wr, b)


def _mid_body(agg_ref, cnt_ref, s_ref, wl_ref, wr_ref, b_ref, y_ref, s2_ref):
    a = agg_ref[0] + agg_ref[1]
    cn = cnt_ref[0, :, 0:1] + cnt_ref[1, :, 0:1]
    rinv = 1.0 / jnp.maximum(cn, 1.0)
    z = jnp.maximum(a * rinv + s_ref[...], 0.0)
    y_ref[...] = jnp.dot(z, wl_ref[...], preferred_element_type=jnp.float32)
    s2_ref[...] = (jnp.dot(z, wr_ref[...], preferred_element_type=jnp.float32)
                   + b_ref[...])


def _mid(agg, cnt, s1, wl, wr, b):
    return pl.pallas_call(
        _mid_body,
        grid=(_NG,),
        in_specs=[_agg_blk, _cnt_blk, _row_blk, _full_w, _full_w, _full_b],
        out_specs=[_row_blk, _row_blk],
        out_shape=[jax.ShapeDtypeStruct((N, D), jnp.float32),
                   jax.ShapeDtypeStruct((N, D), jnp.float32)],
    )(agg, cnt, s1, wl, wr, b)


def _fin_body(agg_ref, cnt_ref, s_ref, o_ref):
    a = agg_ref[0] + agg_ref[1]
    cn = cnt_ref[0, :, 0:1] + cnt_ref[1, :, 0:1]
    rinv = 1.0 / jnp.maximum(cn, 1.0)
    o_ref[...] = a * rinv + s_ref[...]


def _fin(agg, cnt, s2):
    return pl.pallas_call(
        _fin_body,
        grid=(_NG,),
        in_specs=[_agg_blk, _cnt_blk, _row_blk],
        out_specs=_row_blk,
        out_shape=jax.ShapeDtypeStruct((N, D), jnp.float32),
    )(agg, cnt, s2)


@jax.jit
def kernel(x, edge_index, W_l1, b_l1, W_r1, W_l2, b_l2, W_r2):
    src = edge_index[0].reshape(NW, NCHUNK, K)
    dst = edge_index[1].reshape(NW, NCHUNK, K)
    y1, s1 = _pre(x, W_l1, W_r1, b_l1.reshape(1, D))
    agg1, cnt = _agg_with_counts(y1, src, dst)
    y2, s2 = _mid(agg1, cnt, s1, W_l2, W_r2, b_l2.reshape(1, D))
    (agg2,) = _agg_no_counts(y2, src, dst)
    return _fin(agg2, cnt, s2)


# speedup vs baseline: 1.1843x; 1.0007x over previous
import functools

import jax
import jax.numpy as jnp
from jax import lax
from jax.experimental import pallas as pl
from jax.experimental.pallas import tpu as pltpu
from jax.experimental.pallas import tpu_sc as plsc

N = 10000
E = 320000
D = 128

NC = 2
NS = 16
NW = NC * NS
EPW = E // NW
K = 40
NCHUNK = EPW // K
QC = 25
NP = 10240
RPT = NP // NS
CW = 16


def _agg_body(with_counts, ahead, nb, *refs):
    if with_counts:
        (y_hbm, src_hbm, dst_hbm, out_hbm, cnt_hbm,
         sidx, didx, rows, acc) = refs[:9]
        gsems = refs[9:9 + ahead]
        (isem, ssa, ssb, csa, csb, ones, czbuf, cacc) = refs[9 + ahead:]
    else:
        (y_hbm, src_hbm, dst_hbm, out_hbm,
         sidx, didx, rows, acc) = refs[:8]
        gsems = refs[8:8 + ahead]
        (isem, ssa, ssb) = refs[8 + ahead:]

    core = lax.axis_index("c")
    sub = lax.axis_index("s")
    wid = core * NS + sub

    pltpu.async_copy(src_hbm.at[wid, pl.ds(0, QC)], sidx.at[0], isem)
    pltpu.async_copy(dst_hbm.at[wid, pl.ds(0, QC)], didx.at[0], isem)

    zero16 = jnp.zeros((16,), jnp.float32)

    for b in range(2):
        def zrow(i, c, _b=b):
            for j in range(D // 16):
                rows[_b, i, pl.ds(j * 16, 16)] = zero16
            return c
        lax.fori_loop(0, K, zrow, 0)

    r0 = sub * RPT
    for t in range(RPT // K):
        @pl.when(r0 + t * K < N)
        def _(_t=t):
            pltpu.sync_copy(rows.at[_t % 2], acc.at[pl.ds(r0 + _t * K, K)])

    if with_counts:
        one16 = jnp.ones((16,), jnp.float32)

        def crow(i, c):
            czbuf[i, :] = zero16
            return c
        lax.fori_loop(0, K, crow, 0)
        for t in range(RPT // K):
            @pl.when(r0 + t * K < N)
            def _(_t=t):
                pltpu.sync_copy(czbuf, cacc.at[pl.ds(r0 + _t * K, K)])

        def orow(i, c):
            ones[i, :] = one16
            return c
        lax.fori_loop(0, K, orow, 0)

    pltpu.make_async_copy(src_hbm.at[wid, pl.ds(0, QC)], sidx.at[0],
                          isem).wait()
    pltpu.make_async_copy(dst_hbm.at[wid, pl.ds(0, QC)], didx.at[0],
                          isem).wait()
    for p in range(ahead):
        pltpu.async_copy(y_hbm.at[sidx.at[0, p]], rows.at[p], gsems[p])

    plsc.subcore_barrier()

    NB = nb
    AHEAD = ahead

    def half(c, gsem, ssem, csem):
        q = lax.div(c, QC)
        r = lax.rem(c, QC)
        qp = lax.rem(q, 2)
        qn = lax.rem(q + 1, 2)
        b0 = lax.rem(c, NB)
        b4 = lax.rem(c + AHEAD, NB)

        pltpu.make_async_copy(y_hbm.at[sidx.at[qp, r]], rows.at[b0],
                              gsem).wait()

        @pl.when(c >= 2)
        def _():
            pltpu.make_async_copy(y_hbm.at[pl.ds(0, K)], rows.at[b0],
                                  ssem).wait()
            if with_counts:
                pltpu.make_async_copy(cnt_hbm.at[0, pl.ds(0, K)], ones,
                                      csem).wait()

        @pl.when((r == 1) & ((q + 1) * QC < NCHUNK))
        def _():
            pltpu.async_copy(src_hbm.at[wid, pl.ds((q + 1) * QC, QC)],
                             sidx.at[qn], isem)
            pltpu.async_copy(dst_hbm.at[wid, pl.ds((q + 1) * QC, QC)],
                             didx.at[qn], isem)

        @pl.when((r == QC - AHEAD) & (c + AHEAD < NCHUNK))
        def _():
            pltpu.make_async_copy(src_hbm.at[wid, pl.ds(0, QC)],
                                  sidx.at[qn], isem).wait()
            pltpu.make_async_copy(dst_hbm.at[wid, pl.ds(0, QC)],
                                  didx.at[qn], isem).wait()

        @pl.when(c + AHEAD < NCHUNK)
        def _():
            q2 = lax.div(c + AHEAD, QC)
            r2 = lax.rem(c + AHEAD, QC)
            qp2 = lax.rem(q2, 2)
            pltpu.async_copy(y_hbm.at[sidx.at[qp2, r2]], rows.at[b4], gsem)

        pltpu.async_copy(rows.at[b0], acc.at[didx.at[qp, r]], ssem, add=True)
        if with_counts:
            pltpu.async_copy(ones, cacc.at[didx.at[qp, r]], csem, add=True)

    _csa = csa if with_counts else None
    _csb = csb if with_counts else None

    def group(i, carry):
        for p in range(AHEAD):
            half(AHEAD * i + p, gsems[p], ssa if p % 2 == 0 else ssb,
                 _csa if p % 2 == 0 else _csb)
        return carry
    lax.fori_loop(0, NCHUNK // AHEAD, group, 0)
    for p in range(NCHUNK % AHEAD):
        c_tail = (NCHUNK // AHEAD) * AHEAD + p
        half(c_tail, gsems[c_tail % AHEAD], ssa if p % 2 == 0 else ssb,
             _csa if p % 2 == 0 else _csb)

    for ss, cs in ((ssa, _csa), (ssb, _csb)):
        pltpu.make_async_copy(y_hbm.at[pl.ds(0, K)], rows.at[0], ss).wait()
        if with_counts:
            pltpu.make_async_copy(cnt_hbm.at[0, pl.ds(0, K)], ones, cs).wait()

    plsc.subcore_barrier()

    LAST = N - (NS - 1) * RPT

    @pl.when(sub < NS - 1)
    def _():
        pltpu.sync_copy(acc.at[pl.ds(r0, RPT)],
                        out_hbm.at[core, pl.ds(r0, RPT)])
        if with_counts:
            pltpu.sync_copy(cacc.at[pl.ds(r0, RPT)],
                            cnt_hbm.at[core, pl.ds(r0, RPT)])

    @pl.when(sub == NS - 1)
    def _():
        pltpu.sync_copy(acc.at[pl.ds((NS - 1) * RPT, LAST)],
                        out_hbm.at[core, pl.ds((NS - 1) * RPT, LAST)])
        if with_counts:
            pltpu.sync_copy(cacc.at[pl.ds((NS - 1) * RPT, LAST)],
                            cnt_hbm.at[core, pl.ds((NS - 1) * RPT, LAST)])


def _make_agg(with_counts, ahead, nb):
    mesh = plsc.VectorSubcoreMesh(core_axis_name="c", subcore_axis_name="s")
    out_type = [jax.ShapeDtypeStruct((NC, NP, D), jnp.float32)]
    scratch = [
        pltpu.VMEM((2, QC, K), jnp.int32),
        pltpu.VMEM((2, QC, K), jnp.int32),
        pltpu.VMEM((nb, K, D), jnp.float32),
        pltpu.VMEM_SHARED((NP, D), jnp.float32),
    ] + [pltpu.SemaphoreType.DMA] * ahead + [
        pltpu.SemaphoreType.DMA,
        pltpu.SemaphoreType.DMA,
        pltpu.SemaphoreType.DMA,
    ]
    if with_counts:
        out_type.append(jax.ShapeDtypeStruct((NC, NP, CW), jnp.float32))
        scratch += [
            pltpu.SemaphoreType.DMA,
            pltpu.SemaphoreType.DMA,
            pltpu.VMEM((K, CW), jnp.float32),
            pltpu.VMEM((K, CW), jnp.float32),
            pltpu.VMEM_SHARED((NP, CW), jnp.float32),
        ]
    return pl.kernel(
        functools.partial(_agg_body, with_counts, ahead, nb),
        out_type=out_type,
        mesh=mesh,
        scratch_types=scratch,
        compiler_params=pltpu.CompilerParams(use_tc_tiling_on_sc=False),
    )


@functools.cache
def _agg_with_counts():
    return _make_agg(True, 4, 6)


@functools.cache
def _agg_no_counts():
    return _make_agg(False, 6, 8)


_RB = 1000
_NG = N // _RB

_full_w = pl.BlockSpec((D, D), lambda i: (0, 0))
_full_b = pl.BlockSpec((1, D), lambda i: (0, 0))
_row_blk = pl.BlockSpec((_RB, D), lambda i: (i, 0))
_agg_blk = pl.BlockSpec((NC, _RB, D), lambda i: (0, i, 0))
_cnt_blk = pl.BlockSpec((NC, _RB, CW), lambda i: (0, i, 0))


def _pre_body(x_ref, wl_ref, wr_ref, b_ref, y_ref, s_ref):
    xb = x_ref[...]
    y_ref[...] = jnp.dot(xb, wl_ref[...], preferred_element_type=jnp.float32)
    s_ref[...] = (jnp.dot(xb, wr_ref[...], preferred_element_type=jnp.float32)
                  + b_ref[...])


def _pre(x, wl, wr, b):
    return pl.pallas_call(
        _pre_body,
        grid=(_NG,),
        in_specs=[_row_blk, _full_w, _full_w, _full_b],
        out_specs=[_row_blk, _row_blk],
        out_shape=[jax.ShapeDtypeStruct((N, D), jnp.float32),
                   jax.ShapeDtypeStruct((N, D), jnp.float32)],
    )(x, wl, wr, b)


def _mid_body(agg_ref, cnt_ref, s_ref, wl_ref, wr_ref, b_ref, y_ref, s2_ref):
    a = agg_ref[0] + agg_ref[1]
    cn = cnt_ref[0, :, 0:1] + cnt_ref[1, :, 0:1]
    rinv = 1.0 / jnp.maximum(cn, 1.0)
    z = jnp.maximum(a * rinv + s_ref[...], 0.0)
    y_ref[...] = jnp.dot(z, wl_ref[...], preferred_element_type=jnp.float32)
    s2_ref[...] = (jnp.dot(z, wr_ref[...], preferred_element_type=jnp.float32)
                   + b_ref[...])


def _mid(agg, cnt, s1, wl, wr, b):
    return pl.pallas_call(
        _mid_body,
        grid=(_NG,),
        in_specs=[_agg_blk, _cnt_blk, _row_blk, _full_w, _full_w, _full_b],
        out_specs=[_row_blk, _row_blk],
        out_shape=[jax.ShapeDtypeStruct((N, D), jnp.float32),
                   jax.ShapeDtypeStruct((N, D), jnp.float32)],
    )(agg, cnt, s1, wl, wr, b)


def _fin_body(agg_ref, cnt_ref, s_ref, o_ref):
    a = agg_ref[0] + agg_ref[1]
    cn = cnt_ref[0, :, 0:1] + cnt_ref[1, :, 0:1]
    rinv = 1.0 / jnp.maximum(cn, 1.0)
    o_ref[...] = a * rinv + s_ref[...]


def _fin(agg, cnt, s2):
    return pl.pallas_call(
        _fin_body,
        grid=(_NG,),
        in_specs=[_agg_blk, _cnt_blk, _row_blk],
        out_specs=_row_blk,
        out_shape=jax.ShapeDtypeStruct((N, D), jnp.float32),
    )(agg, cnt, s2)


@jax.jit
def kernel(x, edge_index, W_l1, b_l1, W_r1, W_l2, b_l2, W_r2):
    src = edge_index[0].reshape(NW, NCHUNK, K)
    dst = edge_index[1].reshape(NW, NCHUNK, K)
    y1, s1 = _pre(x, W_l1, W_r1, b_l1.reshape(1, D))
    agg1, cnt = _agg_with_counts()(y1, src, dst)
    y2, s2 = _mid(agg1, cnt, s1, W_l2, W_r2, b_l2.reshape(1, D))
    (agg2,) = _agg_no_counts()(y2, src, dst)
    return _fin(agg2, cnt, s2)
